# baseline (speedup 1.0000x reference)
import functools

import jax
import jax.numpy as jnp
from jax import lax
from jax.experimental import pallas as pl
from jax.experimental.pallas import tpu as pltpu
from jax.experimental.pallas import tpu_sc as plsc

N = 10000
E = 320000
D_IN = 128
D_OUT = 256

NC = 2
NS = 16
L = 16
NW = NC * NS
CHUNK = 64
DCHUNK = 64
EPT_PAD = 10240
NCH = EPT_PAD // CHUNK
NCH_H = NCH // 4
DNCH = EPT_PAD // DCHUNK
DNCH_H = DNCH // 2
N_PAD = 10240
N_PER_TILE = N_PAD // NS
DEG_W = 128

_mesh = plsc.VectorSubcoreMesh(core_axis_name="c", subcore_axis_name="s")


def _bcast_lane(vec, lane):
  idx = jnp.full((L, 1), lane, jnp.int32)
  dn = lax.GatherDimensionNumbers(
      offset_dims=(), collapsed_slice_dims=(0,), start_index_map=(0,))
  return lax.gather(vec, idx, dn, (1,),
                    mode=lax.GatherScatterMode.PROMISE_IN_BOUNDS)


@functools.partial(
    pl.kernel,
    out_type=jax.ShapeDtypeStruct((NC, N_PAD), jnp.float32),
    mesh=_mesh,
    compiler_params=pltpu.CompilerParams(needs_layout_passes=False),
    scratch_types=[
        pltpu.VMEM((DNCH, DCHUNK), jnp.int32),
        pltpu.VMEM((DNCH, DCHUNK), jnp.float32),
        pltpu.VMEM_SHARED((N_PAD,), jnp.float32),
        pltpu.SemaphoreType.DMA,
    ],
)
def _deg_kernel(row_hbm, ew_hbm, zeros_hbm, deg_hbm,
                ridx_all, ew_all, deg_sh, sem0):
  c = lax.axis_index("c")
  s = lax.axis_index("s")
  wid = s * NC + c
  pltpu.sync_copy(zeros_hbm.at[0, pl.ds(s * N_PER_TILE, N_PER_TILE)],
                  deg_sh.at[pl.ds(s * N_PER_TILE, N_PER_TILE)])
  pltpu.sync_copy(row_hbm.at[wid], ridx_all)
  pltpu.sync_copy(ew_hbm.at[wid], ew_all)
  plsc.subcore_barrier()

  def body(i, carry):
    pltpu.async_copy(ew_all.at[i], deg_sh.at[ridx_all.at[i]], sem0,
                     add=True)
    pltpu.make_async_copy(ew_all.at[i], deg_sh.at[ridx_all.at[i]],
                          sem0).wait()
    return carry

  lax.fori_loop(0, DNCH, body, 0)
  plsc.subcore_barrier()
  pltpu.sync_copy(deg_sh.at[pl.ds(s * N_PER_TILE, N_PER_TILE)],
                  deg_hbm.at[c, pl.ds(s * N_PER_TILE, N_PER_TILE)])


@functools.partial(
    pl.kernel,
    out_type=jax.ShapeDtypeStruct((NC, N_PAD, D_IN), jnp.float32),
    mesh=_mesh,
    compiler_params=pltpu.CompilerParams(needs_layout_passes=False),
    scratch_types=[
        pltpu.VMEM((NCH_H, CHUNK), jnp.int32),
        pltpu.VMEM((NCH_H, CHUNK), jnp.int32),
        pltpu.VMEM((NCH_H, CHUNK), jnp.float32),
        pltpu.VMEM((CHUNK, D_IN), jnp.float32),
        pltpu.VMEM((CHUNK, D_IN), jnp.float32),
        pltpu.VMEM((CHUNK, D_IN), jnp.float32),
        pltpu.VMEM((CHUNK, D_IN), jnp.float32),
        pltpu.VMEM_SHARED((N_PAD, D_IN), jnp.float32),
        pltpu.SemaphoreType.DMA,
        pltpu.SemaphoreType.DMA,
        pltpu.SemaphoreType.DMA,
        pltpu.SemaphoreType.DMA,
    ],
)
def _agg_kernel(col_hbm, row_hbm, ew_hbm, y_hbm, zeros_hbm, agg_hbm,
                cidx_all, ridx_all, ew_all, rows0, rows1, rows2, rows3,
                agg_sh, sem0, sem1, sem2, sem3):
  c = lax.axis_index("c")
  s = lax.axis_index("s")
  wid = s * NC + c
  pltpu.sync_copy(zeros_hbm, agg_sh.at[pl.ds(s * N_PER_TILE, N_PER_TILE), :])
  plsc.subcore_barrier()

  def start_gather(buf, ci, sem):
    pltpu.async_copy(y_hbm.at[cidx_all.at[ci]], buf, sem)

  def wait_gather(buf, ci, sem):
    pltpu.make_async_copy(y_hbm.at[cidx_all.at[ci]], buf, sem).wait()

  def scale(buf, ci):
    def g_body(g, carry):
      ew16 = ew_all[ci, pl.ds(g * L, L)]
      for l in range(L):
        sbc = _bcast_lane(ew16, l)
        for d in range(D_IN // L):
          buf[g * L + l, pl.ds(d * L, L)] = (
              buf[g * L + l, pl.ds(d * L, L)] * sbc)
      return carry
    lax.fori_loop(0, CHUNK // L, g_body, 0)

  bufs = (rows0, rows1, rows2, rows3)
  sems = (sem0, sem1, sem2, sem3)
  NB = 4
  for h in range(4):
    pltpu.sync_copy(col_hbm.at[wid, pl.ds(h * NCH_H, NCH_H), :], cidx_all)
    pltpu.sync_copy(row_hbm.at[wid, pl.ds(h * NCH_H, NCH_H), :], ridx_all)
    pltpu.sync_copy(ew_hbm.at[wid, pl.ds(h * NCH_H, NCH_H), :], ew_all)
    for j in range(NB):
      start_gather(bufs[j], j, sems[j])

    def body(i, carry):
      for j in range(NB):
        cj = NB * i + j
        wait_gather(bufs[j], cj, sems[j])
        scale(bufs[j], cj)
        pltpu.sync_copy(bufs[j], agg_sh.at[ridx_all.at[cj]], add=True)

        @pl.when(cj + NB < NCH_H)
        def _():
          start_gather(bufs[j], cj + NB, sems[j])
      return carry

    lax.fori_loop(0, NCH_H // NB, body, 0)
  plsc.subcore_barrier()
  pltpu.sync_copy(agg_sh.at[pl.ds(s * N_PER_TILE, N_PER_TILE), :],
                  agg_hbm.at[c, pl.ds(s * N_PER_TILE, N_PER_TILE), :])


R_BLK = 1024


def _y_body(deg_ref, x_ref, y_ref):
  d = deg_ref[...]
  deg = (d[0] + d[1] + 1.0)[:, None]
  dis = jnp.where(deg > 0.0, lax.rsqrt(deg), 0.0)
  y_ref[...] = x_ref[...] * dis


def _out_body(agg_ref, deg_ref, x_ref, w_ref, o_ref):
  d = deg_ref[...]
  deg = (d[0] + d[1] + 1.0)[:, None]
  dis = jnp.where(deg > 0.0, lax.rsqrt(deg), 0.0)
  h = (agg_ref[0] + agg_ref[1]) * dis + x_ref[...] * (dis * dis)
  o_ref[...] = jnp.dot(h, w_ref[...], preferred_element_type=jnp.float32)


_y_call = pl.pallas_call(
    _y_body,
    grid=(pl.cdiv(N, R_BLK),),
    in_specs=[
        pl.BlockSpec((NC, R_BLK), lambda i: (0, i)),
        pl.BlockSpec((R_BLK, D_IN), lambda i: (i, 0)),
    ],
    out_specs=pl.BlockSpec((R_BLK, D_IN), lambda i: (i, 0)),
    out_shape=jax.ShapeDtypeStruct((N, D_IN), jnp.float32),
)

_out_call = pl.pallas_call(
    _out_body,
    grid=(pl.cdiv(N, R_BLK),),
    in_specs=[
        pl.BlockSpec((NC, R_BLK, D_IN), lambda i: (0, i, 0)),
        pl.BlockSpec((NC, R_BLK), lambda i: (0, i)),
        pl.BlockSpec((R_BLK, D_IN), lambda i: (i, 0)),
        pl.BlockSpec((D_IN, D_OUT), lambda i: (0, 0)),
    ],
    out_specs=pl.BlockSpec((R_BLK, D_OUT), lambda i: (i, 0)),
    out_shape=jax.ShapeDtypeStruct((N, D_OUT), jnp.float32),
)


@jax.jit
def kernel(x, edge_index, edge_weight, W):
  ei = edge_index.astype(jnp.int32)
  pad = EPT_PAD - E // NW
  row_p = jnp.pad(ei[0].reshape(NW, E // NW), ((0, 0), (0, pad)))
  col_p = jnp.pad(ei[1].reshape(NW, E // NW), ((0, 0), (0, pad)))
  ew_p = jnp.pad(edge_weight.reshape(NW, E // NW), ((0, 0), (0, pad)))
  zeros_agg = jnp.zeros((N_PER_TILE, D_IN), jnp.float32)
  zeros_deg = jnp.zeros((1, N_PAD), jnp.float32)
  deg2d = _deg_kernel(row_p.reshape(NW, DNCH, DCHUNK),
                      ew_p.reshape(NW, DNCH, DCHUNK), zeros_deg)[:, :N]
  y = _y_call(deg2d, x)
  agg2d = _agg_kernel(col_p.reshape(NW, NCH, CHUNK),
                      row_p.reshape(NW, NCH, CHUNK),
                      ew_p.reshape(NW, NCH, CHUNK), y, zeros_agg)[:, :N, :]
  return _out_call(agg2d, deg2d, x, W)


# speedup vs baseline: 1.0108x; 1.0108x over previous
import functools

import jax
import jax.numpy as jnp
from jax import lax
from jax.experimental import pallas as pl
from jax.experimental.pallas import tpu as pltpu
from jax.experimental.pallas import tpu_sc as plsc

N = 10000
E = 320000
D_IN = 128
D_OUT = 256

NC = 2
NS = 16
L = 16
NW = NC * NS
CHUNK = 128
DCHUNK = 64
EPT_PAD = 10240
NCH = EPT_PAD // CHUNK
NCH_H = NCH // 2
DNCH = EPT_PAD // DCHUNK
DNCH_H = DNCH // 2
N_PAD = 10240
N_PER_TILE = N_PAD // NS
DEG_W = 128

_mesh = plsc.VectorSubcoreMesh(core_axis_name="c", subcore_axis_name="s")


def _bcast_lane(vec, lane):
  idx = jnp.full((L, 1), lane, jnp.int32)
  dn = lax.GatherDimensionNumbers(
      offset_dims=(), collapsed_slice_dims=(0,), start_index_map=(0,))
  return lax.gather(vec, idx, dn, (1,),
                    mode=lax.GatherScatterMode.PROMISE_IN_BOUNDS)


@functools.partial(
    pl.kernel,
    out_type=jax.ShapeDtypeStruct((NC, N_PAD), jnp.float32),
    mesh=_mesh,
    compiler_params=pltpu.CompilerParams(needs_layout_passes=False),
    scratch_types=[
        pltpu.VMEM((DNCH, DCHUNK), jnp.int32),
        pltpu.VMEM((DNCH, DCHUNK), jnp.float32),
        pltpu.VMEM_SHARED((N_PAD,), jnp.float32),
        pltpu.SemaphoreType.DMA,
    ],
)
def _deg_kernel(row_hbm, ew_hbm, zeros_hbm, deg_hbm,
                ridx_all, ew_all, deg_sh, sem0):
  c = lax.axis_index("c")
  s = lax.axis_index("s")
  wid = s * NC + c
  pltpu.sync_copy(zeros_hbm.at[0, pl.ds(s * N_PER_TILE, N_PER_TILE)],
                  deg_sh.at[pl.ds(s * N_PER_TILE, N_PER_TILE)])
  pltpu.sync_copy(row_hbm.at[wid], ridx_all)
  pltpu.sync_copy(ew_hbm.at[wid], ew_all)
  plsc.subcore_barrier()

  def body(i, carry):
    pltpu.async_copy(ew_all.at[i], deg_sh.at[ridx_all.at[i]], sem0,
                     add=True)
    pltpu.make_async_copy(ew_all.at[i], deg_sh.at[ridx_all.at[i]],
                          sem0).wait()
    return carry

  lax.fori_loop(0, DNCH, body, 0)
  plsc.subcore_barrier()
  pltpu.sync_copy(deg_sh.at[pl.ds(s * N_PER_TILE, N_PER_TILE)],
                  deg_hbm.at[c, pl.ds(s * N_PER_TILE, N_PER_TILE)])


@functools.partial(
    pl.kernel,
    out_type=jax.ShapeDtypeStruct((NC, N_PAD, D_IN), jnp.float32),
    mesh=_mesh,
    compiler_params=pltpu.CompilerParams(needs_layout_passes=False),
    scratch_types=[
        pltpu.VMEM((NCH_H, CHUNK), jnp.int32),
        pltpu.VMEM((NCH_H, CHUNK), jnp.int32),
        pltpu.VMEM((NCH_H, CHUNK), jnp.float32),
        pltpu.VMEM((CHUNK, D_IN), jnp.float32),
        pltpu.VMEM((CHUNK, D_IN), jnp.float32),
        pltpu.VMEM_SHARED((N_PAD, D_IN), jnp.float32),
        pltpu.SemaphoreType.DMA,
        pltpu.SemaphoreType.DMA,
    ],
)
def _agg_kernel(col_hbm, row_hbm, ew_hbm, y_hbm, zeros_hbm, agg_hbm,
                cidx_all, ridx_all, ew_all, rows0, rows1, agg_sh,
                sem0, sem1):
  c = lax.axis_index("c")
  s = lax.axis_index("s")
  wid = s * NC + c
  pltpu.sync_copy(zeros_hbm, agg_sh.at[pl.ds(s * N_PER_TILE, N_PER_TILE), :])
  plsc.subcore_barrier()

  def start_gather(buf, ci, sem):
    pltpu.async_copy(y_hbm.at[cidx_all.at[ci]], buf, sem)

  def wait_gather(buf, ci, sem):
    pltpu.make_async_copy(y_hbm.at[cidx_all.at[ci]], buf, sem).wait()

  def scale(buf, ci):
    def g_body(g, carry):
      ew16 = ew_all[ci, pl.ds(g * L, L)]
      for l in range(L):
        sbc = _bcast_lane(ew16, l)
        for d in range(D_IN // L):
          buf[g * L + l, pl.ds(d * L, L)] = (
              buf[g * L + l, pl.ds(d * L, L)] * sbc)
      return carry
    lax.fori_loop(0, CHUNK // L, g_body, 0)

  for h in range(2):
    pltpu.sync_copy(col_hbm.at[wid, pl.ds(h * NCH_H, NCH_H), :], cidx_all)
    pltpu.sync_copy(row_hbm.at[wid, pl.ds(h * NCH_H, NCH_H), :], ridx_all)
    pltpu.sync_copy(ew_hbm.at[wid, pl.ds(h * NCH_H, NCH_H), :], ew_all)
    start_gather(rows0, 0, sem0)

    def body(i, carry):
      c0 = 2 * i
      c1 = 2 * i + 1
      wait_gather(rows0, c0, sem0)
      start_gather(rows1, c1, sem1)
      scale(rows0, c0)
      pltpu.sync_copy(rows0, agg_sh.at[ridx_all.at[c0]], add=True)
      wait_gather(rows1, c1, sem1)

      @pl.when(i < NCH_H // 2 - 1)
      def _():
        start_gather(rows0, c0 + 2, sem0)

      scale(rows1, c1)
      pltpu.sync_copy(rows1, agg_sh.at[ridx_all.at[c1]], add=True)
      return carry

    lax.fori_loop(0, NCH_H // 2, body, 0)
  plsc.subcore_barrier()
  pltpu.sync_copy(agg_sh.at[pl.ds(s * N_PER_TILE, N_PER_TILE), :],
                  agg_hbm.at[c, pl.ds(s * N_PER_TILE, N_PER_TILE), :])


R_BLK = 1024


def _y_body(deg_ref, x_ref, y_ref):
  d = deg_ref[...]
  deg = (d[0] + d[1] + 1.0)[:, None]
  dis = jnp.where(deg > 0.0, lax.rsqrt(deg), 0.0)
  y_ref[...] = x_ref[...] * dis


def _out_body(agg_ref, deg_ref, x_ref, w_ref, o_ref):
  d = deg_ref[...]
  deg = (d[0] + d[1] + 1.0)[:, None]
  dis = jnp.where(deg > 0.0, lax.rsqrt(deg), 0.0)
  h = (agg_ref[0] + agg_ref[1]) * dis + x_ref[...] * (dis * dis)
  o_ref[...] = jnp.dot(h, w_ref[...], preferred_element_type=jnp.float32)


_y_call = pl.pallas_call(
    _y_body,
    grid=(pl.cdiv(N, R_BLK),),
    in_specs=[
        pl.BlockSpec((NC, R_BLK), lambda i: (0, i)),
        pl.BlockSpec((R_BLK, D_IN), lambda i: (i, 0)),
    ],
    out_specs=pl.BlockSpec((R_BLK, D_IN), lambda i: (i, 0)),
    out_shape=jax.ShapeDtypeStruct((N, D_IN), jnp.float32),
)

_out_call = pl.pallas_call(
    _out_body,
    grid=(pl.cdiv(N, R_BLK),),
    in_specs=[
        pl.BlockSpec((NC, R_BLK, D_IN), lambda i: (0, i, 0)),
        pl.BlockSpec((NC, R_BLK), lambda i: (0, i)),
        pl.BlockSpec((R_BLK, D_IN), lambda i: (i, 0)),
        pl.BlockSpec((D_IN, D_OUT), lambda i: (0, 0)),
    ],
    out_specs=pl.BlockSpec((R_BLK, D_OUT), lambda i: (i, 0)),
    out_shape=jax.ShapeDtypeStruct((N, D_OUT), jnp.float32),
)


@jax.jit
def kernel(x, edge_index, edge_weight, W):
  ei = edge_index.astype(jnp.int32)
  pad = EPT_PAD - E // NW
  row_p = jnp.pad(ei[0].reshape(NW, E // NW), ((0, 0), (0, pad)))
  col_p = jnp.pad(ei[1].reshape(NW, E // NW), ((0, 0), (0, pad)))
  ew_p = jnp.pad(edge_weight.reshape(NW, E // NW), ((0, 0), (0, pad)))
  zeros_agg = jnp.zeros((N_PER_TILE, D_IN), jnp.float32)
  zeros_deg = jnp.zeros((1, N_PAD), jnp.float32)
  deg2d = _deg_kernel(row_p.reshape(NW, DNCH, DCHUNK),
                      ew_p.reshape(NW, DNCH, DCHUNK), zeros_deg)[:, :N]
  y = _y_call(deg2d, x)
  agg2d = _agg_kernel(col_p.reshape(NW, NCH, CHUNK),
                      row_p.reshape(NW, NCH, CHUNK),
                      ew_p.reshape(NW, NCH, CHUNK), y, zeros_agg)[:, :N, :]
  return _out_call(agg2d, deg2d, x, W)
